# baseline (speedup 1.0000x reference)
import functools

import jax
import jax.numpy as jnp
from jax import lax
from jax.experimental import pallas as pl
from jax.experimental.pallas import tpu as pltpu
from jax.experimental.pallas import tpu_sc as plsc

MAXLEN = 8192
OUTPUT_DIM = 1024

_info = plsc.get_sparse_core_info()
NC = _info.num_cores
ROWS_PER_W = MAXLEN // NC

CHUNK = 64
NCHUNK = ROWS_PER_W // CHUNK
NBUF = 24

_mesh = plsc.ScalarSubcoreMesh(axis_name="c", num_cores=NC)


@functools.partial(
    pl.kernel,
    mesh=_mesh,
    out_type=jax.ShapeDtypeStruct((MAXLEN, OUTPUT_DIM), jnp.float32),
    scratch_types=[
        pltpu.VMEM_SHARED((NBUF, CHUNK, OUTPUT_DIM), jnp.float32),
        pltpu.SemaphoreType.DMA((NBUF,)),
        pltpu.SemaphoreType.DMA((NBUF,)),
    ],
)
def _sc_copy(table_hbm, out_hbm, buf, in_sem, out_sem):
    base = lax.axis_index("c") * ROWS_PER_W

    def in_copy(c):
        return pltpu.make_async_copy(
            table_hbm.at[pl.ds(base + c * CHUNK, CHUNK), :],
            buf.at[c % NBUF],
            in_sem.at[c % NBUF],
        )

    def out_copy(c):
        return pltpu.make_async_copy(
            buf.at[c % NBUF],
            out_hbm.at[pl.ds(base + c * CHUNK, CHUNK), :],
            out_sem.at[c % NBUF],
        )

    for c in range(NBUF - 1):
        in_copy(c).start()
    for c in range(NCHUNK):
        in_copy(c).wait()
        out_copy(c).start()
        nxt = c + NBUF - 1
        if nxt < NCHUNK:
            if nxt >= NBUF:
                out_copy(nxt - NBUF).wait()
            in_copy(nxt).start()
    for c in range(NCHUNK - NBUF, NCHUNK):
        out_copy(c).wait()


def kernel(inputs, table):
    del inputs
    return _sc_copy(table)[None]


# speedup vs baseline: 1.0878x; 1.0878x over previous
import functools

import jax
import jax.numpy as jnp
from jax import lax
from jax.experimental import pallas as pl
from jax.experimental.pallas import tpu as pltpu
from jax.experimental.pallas import tpu_sc as plsc

MAXLEN = 8192
OUTPUT_DIM = 1024

_info = plsc.get_sparse_core_info()
NC = _info.num_cores
ROWS_PER_W = MAXLEN // NC

CHUNK = 128
NCHUNK = ROWS_PER_W // CHUNK
NBUF = 12

_mesh = plsc.ScalarSubcoreMesh(axis_name="c", num_cores=NC)


@functools.partial(
    pl.kernel,
    mesh=_mesh,
    out_type=jax.ShapeDtypeStruct((MAXLEN, OUTPUT_DIM), jnp.float32),
    scratch_types=[
        pltpu.VMEM_SHARED((NBUF, CHUNK, OUTPUT_DIM), jnp.float32),
        pltpu.SemaphoreType.DMA((NBUF,)),
        pltpu.SemaphoreType.DMA((NBUF,)),
    ],
)
def _sc_copy(table_hbm, out_hbm, buf, in_sem, out_sem):
    base = lax.axis_index("c") * ROWS_PER_W

    def in_copy(c):
        return pltpu.make_async_copy(
            table_hbm.at[pl.ds(base + c * CHUNK, CHUNK), :],
            buf.at[c % NBUF],
            in_sem.at[c % NBUF],
        )

    def out_copy(c):
        return pltpu.make_async_copy(
            buf.at[c % NBUF],
            out_hbm.at[pl.ds(base + c * CHUNK, CHUNK), :],
            out_sem.at[c % NBUF],
        )

    for c in range(NBUF - 1):
        in_copy(c).start()
    for c in range(NCHUNK):
        in_copy(c).wait()
        out_copy(c).start()
        nxt = c + NBUF - 1
        if nxt < NCHUNK:
            if nxt >= NBUF:
                out_copy(nxt - NBUF).wait()
            in_copy(nxt).start()
    for c in range(NCHUNK - NBUF, NCHUNK):
        out_copy(c).wait()


def kernel(inputs, table):
    del inputs
    return _sc_copy(table)[None]
